# baseline (speedup 1.0000x reference)
import functools

import jax
import jax.numpy as jnp
from jax import lax
from jax.experimental import pallas as pl
from jax.experimental.pallas import tpu as pltpu
from jax.experimental.pallas import tpu_sc as plsc

B, H, W = 32, 1024, 1024
P = 128
L = 16
LANES = 128
SUBL = 8


def kernel(x, probe_x, probe_y):
    n_tile_rows = B * H // SUBL
    xg = (
        x.reshape(n_tile_rows, SUBL, W // LANES, LANES)
        .transpose(0, 2, 1, 3)
        .reshape(B * H * W)
    )
    mesh = plsc.VectorSubcoreMesh(
        core_axis_name="c", subcore_axis_name="s", num_cores=1
    )

    @functools.partial(
        pl.kernel,
        mesh=mesh,
        out_type=jax.ShapeDtypeStruct((B, P), jnp.float32),
        scratch_types=[
            pltpu.VMEM((P,), jnp.int32),
            pltpu.VMEM((P,), jnp.int32),
            pltpu.VMEM((2 * P,), jnp.int32),
            pltpu.VMEM((2, P), jnp.float32),
            pltpu.SemaphoreType.DMA,
            pltpu.SemaphoreType.DMA,
            pltpu.SemaphoreType.DMA,
        ],
        compiler_params=pltpu.CompilerParams(
            needs_layout_passes=False, skip_device_barrier=True
        ),
    )
    def probe_gather(x_hbm, px_hbm, py_hbm, out_hbm, px_v, py_v, row_v, val_v,
                     sem_px, sem_py, sem_g):
        wid = lax.axis_index("s")
        cp_px = pltpu.async_copy(px_hbm, px_v, sem_px)
        cp_py = pltpu.async_copy(py_hbm, py_v, sem_py)
        cp_px.wait()
        cp_py.wait()
        base = 2 * wid * (H * W)
        half = P // 2
        fires = []
        for h in range(2):
            for i in range(half // L):
                sl = pl.ds(h * half + i * L, L)
                px, py = px_v[sl], py_v[sl]
                row_v[sl] = (
                    base
                    + (px >> 3) * (SUBL * W)
                    + (py >> 7) * (SUBL * LANES)
                    + (px & 7) * LANES
                    + (py & 127)
                )
            fires.append(pltpu.async_copy(
                x_hbm.at[row_v.at[pl.ds(h * half, half)]],
                val_v.at[0, pl.ds(h * half, half)],
                sem_g,
            ))
        for h in range(2):
            for i in range(half // L):
                sl = pl.ds(h * half + i * L, L)
                row_v[pl.ds(P + h * half + i * L, L)] = row_v[sl] + (H * W)
            fires.append(pltpu.async_copy(
                x_hbm.at[row_v.at[pl.ds(P + h * half, half)]],
                val_v.at[1, pl.ds(h * half, half)],
                sem_g,
            ))
        for cp in fires:
            cp.wait()
        pltpu.sync_copy(val_v, out_hbm.at[pl.ds(2 * wid, 2)])

    return probe_gather(xg, probe_x, probe_y)


# speedup vs baseline: 1.0045x; 1.0045x over previous
import functools

import jax
import jax.numpy as jnp
from jax import lax
from jax.experimental import pallas as pl
from jax.experimental.pallas import tpu as pltpu
from jax.experimental.pallas import tpu_sc as plsc

B, H, W = 32, 1024, 1024
P = 128
L = 16
LANES = 128
SUBL = 8


def kernel(x, probe_x, probe_y):
    n_tile_rows = B * H // SUBL
    xg = (
        x.reshape(n_tile_rows, SUBL, W // LANES, LANES)
        .transpose(0, 2, 1, 3)
        .reshape(B * H * W)
    )
    mesh = plsc.VectorSubcoreMesh(
        core_axis_name="c", subcore_axis_name="s", num_cores=1
    )

    @functools.partial(
        pl.kernel,
        mesh=mesh,
        out_type=jax.ShapeDtypeStruct((B, P), jnp.float32),
        scratch_types=[
            pltpu.VMEM((P,), jnp.int32),
            pltpu.VMEM((P,), jnp.int32),
            pltpu.VMEM((2 * P,), jnp.int32),
            pltpu.VMEM((2, P), jnp.float32),
            pltpu.SemaphoreType.DMA,
            pltpu.SemaphoreType.DMA,
            pltpu.SemaphoreType.DMA,
        ],
        compiler_params=pltpu.CompilerParams(
            needs_layout_passes=False, skip_device_barrier=True
        ),
    )
    def probe_gather(x_hbm, px_hbm, py_hbm, out_hbm, px_v, py_v, row_v, val_v,
                     sem_px, sem_py, sem_g):
        wid = lax.axis_index("s")
        cp_px = pltpu.async_copy(px_hbm, px_v, sem_px)
        cp_py = pltpu.async_copy(py_hbm, py_v, sem_py)
        cp_px.wait()
        cp_py.wait()
        base = 2 * wid * (H * W)
        for i in range(P // L):
            sl = pl.ds(i * L, L)
            px, py = px_v[sl], py_v[sl]
            row_v[sl] = (
                base
                + (px >> 3) * (SUBL * W)
                + (py >> 7) * (SUBL * LANES)
                + (px & 7) * LANES
                + (py & 127)
            )
        cp0 = pltpu.async_copy(
            x_hbm.at[row_v.at[pl.ds(0, P)]], val_v.at[0], sem_g
        )
        for i in range(P // L):
            sl = pl.ds(i * L, L)
            row_v[pl.ds(P + i * L, L)] = row_v[sl] + (H * W)
        cp1 = pltpu.async_copy(
            x_hbm.at[row_v.at[pl.ds(P, P)]], val_v.at[1], sem_px
        )
        cp0.wait()
        cp1.wait()
        pltpu.sync_copy(val_v, out_hbm.at[pl.ds(2 * wid, 2)])

    return probe_gather(xg, probe_x, probe_y)


# speedup vs baseline: 1.0055x; 1.0010x over previous
import functools

import jax
import jax.numpy as jnp
from jax import lax
from jax.experimental import pallas as pl
from jax.experimental.pallas import tpu as pltpu
from jax.experimental.pallas import tpu_sc as plsc

B, H, W = 32, 1024, 1024
P = 128
L = 16
LANES = 128
SUBL = 8


def kernel(x, probe_x, probe_y):
    n_tile_rows = B * H // SUBL
    xg = (
        x.reshape(n_tile_rows, SUBL, W // LANES, LANES)
        .transpose(0, 2, 1, 3)
        .reshape(B * H * W)
    )
    mesh = plsc.VectorSubcoreMesh(
        core_axis_name="c", subcore_axis_name="s", num_cores=1
    )

    @functools.partial(
        pl.kernel,
        mesh=mesh,
        out_type=jax.ShapeDtypeStruct((B, P), jnp.float32),
        scratch_types=[
            pltpu.VMEM((P,), jnp.int32),
            pltpu.VMEM((P,), jnp.int32),
            pltpu.VMEM((2 * P,), jnp.int32),
            pltpu.VMEM((2, P), jnp.float32),
            pltpu.SemaphoreType.DMA,
            pltpu.SemaphoreType.DMA,
            pltpu.SemaphoreType.DMA,
        ],
    )
    def probe_gather(x_hbm, px_hbm, py_hbm, out_hbm, px_v, py_v, row_v, val_v,
                     sem_px, sem_py, sem_g):
        wid = lax.axis_index("s")
        cp_px = pltpu.async_copy(px_hbm, px_v, sem_px)
        cp_py = pltpu.async_copy(py_hbm, py_v, sem_py)
        cp_px.wait()
        cp_py.wait()
        base = 2 * wid * (H * W)
        for i in range(P // L):
            sl = pl.ds(i * L, L)
            px, py = px_v[sl], py_v[sl]
            row_v[sl] = (
                base
                + (px >> 3) * (SUBL * W)
                + (py >> 7) * (SUBL * LANES)
                + (px & 7) * LANES
                + (py & 127)
            )
        cp0 = pltpu.async_copy(
            x_hbm.at[row_v.at[pl.ds(0, P)]], val_v.at[0], sem_g
        )
        for i in range(P // L):
            sl = pl.ds(i * L, L)
            row_v[pl.ds(P + i * L, L)] = row_v[sl] + (H * W)
        cp1 = pltpu.async_copy(
            x_hbm.at[row_v.at[pl.ds(P, P)]], val_v.at[1], sem_px
        )
        cp0.wait()
        cp1.wait()
        pltpu.sync_copy(val_v, out_hbm.at[pl.ds(2 * wid, 2)])

    return probe_gather(xg, probe_x, probe_y)
